# baseline (speedup 1.0000x reference)
import functools

import jax
import jax.numpy as jnp
from jax import lax
from jax.experimental import pallas as pl
from jax.experimental.pallas import tpu as pltpu
from jax.experimental.pallas import tpu_sc as plsc

B, CIN, COUT, T, E, K = 2, 768, 1024, 2048, 8, 2
N = B * T
NP = N * K
TSG = 256
PT = NP + E * TSG
NT = PT // TSG
TT = 512
NW = 32
CINW = CIN // 2
COUTW = COUT // 2


def _router_body(x_ref, rw_ref, ti_ref, tv_ref, xt_ref, rk_ref, cnt_ref,
                 acc_ref):
    x = x_ref[0]
    logits = jax.lax.dot_general(
        x, rw_ref[...], (((0,), (0,)), ((), ())),
        preferred_element_type=jnp.float32)
    m = jnp.max(logits, axis=-1, keepdims=True)
    p = jnp.exp(logits - m)
    probs = p / jnp.sum(p, axis=-1, keepdims=True)
    eidx = jax.lax.broadcasted_iota(jnp.int32, (TT, E), 1)
    v1 = jnp.max(probs, axis=-1)
    i1 = jnp.min(jnp.where(probs == v1[:, None], eidx, E), axis=-1)
    probs2 = jnp.where(eidx == i1[:, None], -1.0, probs)
    v2 = jnp.max(probs2, axis=-1)
    i2 = jnp.min(jnp.where(probs2 == v2[:, None], eidx, E), axis=-1)
    s = v1 + v2
    ti_ref[0] = jnp.stack([i1, i2])
    tv_ref[0] = jnp.stack([v1 / s, v2 / s])
    xt_ref[0] = pltpu.bitcast(x.astype(jnp.bfloat16), jnp.float32).T

    first = (pl.program_id(0) == 0) & (pl.program_id(1) == 0)

    @pl.when(first)
    def _():
        acc_ref[...] = jnp.zeros((1, E), jnp.float32)

    cnt = acc_ref[0]
    tri = (jax.lax.broadcasted_iota(jnp.int32, (TT, TT), 0)
           > jax.lax.broadcasted_iota(jnp.int32, (TT, TT), 1)
           ).astype(jnp.float32)
    oneh1 = (eidx == i1[:, None]).astype(jnp.float32)
    ex1 = jax.lax.dot_general(tri, oneh1, (((1,), (0,)), ((), ())),
                              preferred_element_type=jnp.float32)
    rank1 = (jnp.sum(ex1 * oneh1, axis=1)
             + jnp.sum(oneh1 * cnt[None, :], axis=1))
    cnt1 = cnt + jnp.sum(oneh1, axis=0)
    oneh2 = (eidx == i2[:, None]).astype(jnp.float32)
    ex2 = jax.lax.dot_general(tri, oneh2, (((1,), (0,)), ((), ())),
                              preferred_element_type=jnp.float32)
    rank2 = (jnp.sum(ex2 * oneh2, axis=1)
             + jnp.sum(oneh2 * cnt1[None, :], axis=1))
    acc_ref[...] = (cnt1 + jnp.sum(oneh2, axis=0))[None, :]
    rk_ref[0] = jnp.stack([rank1, rank2])
    cnt_ref[...] = acc_ref[...]


def _router(x, router_w):
    return pl.pallas_call(
        _router_body,
        grid=(B, T // TT),
        in_specs=[
            pl.BlockSpec((1, CIN, TT), lambda b, t: (b, 0, t)),
            pl.BlockSpec((CIN, E), lambda b, t: (0, 0)),
        ],
        out_specs=[
            pl.BlockSpec((1, K, TT), lambda b, t: (b, 0, t)),
            pl.BlockSpec((1, K, TT), lambda b, t: (b, 0, t)),
            pl.BlockSpec((1, TT, CINW), lambda b, t: (b, t, 0)),
            pl.BlockSpec((1, K, TT), lambda b, t: (b, 0, t)),
            pl.BlockSpec((1, E), lambda b, t: (0, 0)),
        ],
        out_shape=[
            jax.ShapeDtypeStruct((B, K, T), jnp.int32),
            jax.ShapeDtypeStruct((B, K, T), jnp.float32),
            jax.ShapeDtypeStruct((B, T, CINW), jnp.float32),
            jax.ShapeDtypeStruct((B, K, T), jnp.float32),
            jax.ShapeDtypeStruct((1, E), jnp.float32),
        ],
        scratch_shapes=[pltpu.VMEM((1, E), jnp.float32)],
    )(x, router_w)


_RPW2 = PT // NW
_CH2 = 40
_RPW4 = N // NW
_CH4 = 32
_NBUF = 6
_LAG = 4


def _sc_mesh():
    return plsc.VectorSubcoreMesh(core_axis_name="c", subcore_axis_name="s")


def _ring_gather(tasks, table_hbm, idx_v, row_bufs, gsems, wsems, chunk):
    nt = len(tasks)
    gd = [None] * nt
    wd = [None] * nt

    def fire_wb(u):
        gd[u].wait()
        _, out_hbm, ooff = tasks[u]
        wd[u] = pltpu.async_copy(row_bufs[u % _NBUF],
                                 out_hbm.at[pl.ds(ooff, chunk)],
                                 wsems[u % _NBUF])

    for t in range(nt):
        b = t % _NBUF
        if t >= _NBUF:
            wd[t - _NBUF].wait()
        ioff, _, _ = tasks[t]
        gd[t] = pltpu.async_copy(
            table_hbm.at[idx_v.at[pl.ds(ioff, chunk)]], row_bufs[b], gsems[b])
        if t >= _LAG:
            fire_wb(t - _LAG)
    for u in range(max(0, nt - _LAG), nt):
        fire_wb(u)
    for u in range(max(0, nt - _NBUF), nt):
        wd[u].wait()


def _gather_x(tok_pad, xt32):
    @functools.partial(
        pl.kernel, mesh=_sc_mesh(),
        out_type=jax.ShapeDtypeStruct((PT, CINW), jnp.float32),
        scratch_types=(
            [pltpu.VMEM((_RPW2,), jnp.int32)]
            + [pltpu.VMEM((_CH2, CINW), jnp.float32)] * _NBUF
            + [pltpu.SemaphoreType.DMA] * (2 * _NBUF)
        ),
    )
    def k(tok_hbm, xt_hbm, out_hbm, idx_v, r0, r1, r2, r3, r4, r5,
          g0, g1, g2, g3, g4, g5, w0, w1, w2, w3, w4, w5):
        wid = lax.axis_index("s") * 2 + lax.axis_index("c")
        base = wid * _RPW2
        pltpu.sync_copy(tok_hbm.at[pl.ds(base, _RPW2)], idx_v)
        tasks = [(c * _CH2, out_hbm, base + c * _CH2)
                 for c in range(_RPW2 // _CH2)]
        _ring_gather(tasks, xt_hbm, idx_v, [r0, r1, r2, r3, r4, r5],
                     [g0, g1, g2, g3, g4, g5], [w0, w1, w2, w3, w4, w5],
                     _CH2)

    return k(tok_pad, xt32)


def _gather_y(p1, p2, yg32):
    @functools.partial(
        pl.kernel, mesh=_sc_mesh(),
        out_type=[jax.ShapeDtypeStruct((N, COUTW), jnp.float32),
                  jax.ShapeDtypeStruct((N, COUTW), jnp.float32)],
        scratch_types=(
            [pltpu.VMEM((2 * _RPW4,), jnp.int32)]
            + [pltpu.VMEM((_CH4, COUTW), jnp.float32)] * _NBUF
            + [pltpu.SemaphoreType.DMA] * (2 * _NBUF)
        ),
    )
    def k(p1_hbm, p2_hbm, yg_hbm, o1_hbm, o2_hbm, idx_v,
          r0, r1, r2, r3, r4, r5,
          g0, g1, g2, g3, g4, g5, w0, w1, w2, w3, w4, w5):
        wid = lax.axis_index("s") * 2 + lax.axis_index("c")
        base = wid * _RPW4
        pltpu.sync_copy(p1_hbm.at[pl.ds(base, _RPW4)],
                        idx_v.at[pl.ds(0, _RPW4)])
        pltpu.sync_copy(p2_hbm.at[pl.ds(base, _RPW4)],
                        idx_v.at[pl.ds(_RPW4, _RPW4)])
        nch = _RPW4 // _CH4
        tasks = ([(c * _CH4, o1_hbm, base + c * _CH4) for c in range(nch)]
                 + [(_RPW4 + c * _CH4, o2_hbm, base + c * _CH4)
                    for c in range(nch)])
        _ring_gather(tasks, yg_hbm, idx_v, [r0, r1, r2, r3, r4, r5],
                     [g0, g1, g2, g3, g4, g5], [w0, w1, w2, w3, w4, w5],
                     _CH4)

    return k(p1, p2, yg32)


def _mm_body(te_ref, xg_ref, ew_ref, eb_ref, yg_ref):
    xb = pltpu.bitcast(xg_ref[...].T, jnp.bfloat16)
    y = jax.lax.dot_general(
        ew_ref[0], xb, (((1,), (0,)), ((), ())),
        preferred_element_type=jnp.float32)
    y = y + eb_ref[0][0][:, None]
    yb = y.astype(jnp.bfloat16)
    yg_ref[...] = pltpu.bitcast(yb, jnp.float32).T


def _grouped_mm(tile_expert, xg32, ew_b, expert_b):
    grid_spec = pltpu.PrefetchScalarGridSpec(
        num_scalar_prefetch=1,
        grid=(NT,),
        in_specs=[
            pl.BlockSpec((TSG, CINW), lambda g, te: (g, 0)),
            pl.BlockSpec((1, COUT, CIN), lambda g, te: (te[g], 0, 0)),
            pl.BlockSpec((1, 1, COUT), lambda g, te: (te[g], 0, 0)),
        ],
        out_specs=pl.BlockSpec((TSG, COUTW), lambda g, te: (g, 0)),
    )
    return pl.pallas_call(
        _mm_body,
        grid_spec=grid_spec,
        out_shape=jax.ShapeDtypeStruct((PT, COUTW), jnp.float32),
    )(tile_expert, xg32, ew_b, expert_b.reshape(E, 1, COUT))


def _ep_body(resb_ref, tv_ref, o1_ref, o2_ref, out_ref):
    y1 = pltpu.bitcast(o1_ref[...].T, jnp.bfloat16)
    y2 = pltpu.bitcast(o2_ref[...].T, jnp.bfloat16)
    v = tv_ref[0]
    acc = resb_ref[0][:, None] + (
        y1.astype(jnp.float32) * v[0][None, :]
        + y2.astype(jnp.float32) * v[1][None, :])
    out_ref[0] = acc


def _epilogue(res_b, tv, o1, o2):
    nt = T // TT
    return pl.pallas_call(
        _ep_body,
        grid=(B, nt),
        in_specs=[
            pl.BlockSpec((1, COUT), lambda b, t: (0, 0)),
            pl.BlockSpec((1, K, TT), lambda b, t: (b, 0, t)),
            pl.BlockSpec((TT, COUTW), lambda b, t: (b * nt + t, 0)),
            pl.BlockSpec((TT, COUTW), lambda b, t: (b * nt + t, 0)),
        ],
        out_specs=pl.BlockSpec((1, COUT, TT), lambda b, t: (b, 0, t)),
        out_shape=jax.ShapeDtypeStruct((B, COUT, T), jnp.float32),
    )(res_b.reshape(1, COUT), tv, o1, o2)


@jax.jit
def _run(x, router_w, expert_w, expert_b, res_w, res_b):
    ti, tv, xt32, rk, cnt = _router(x, router_w)
    topi = jnp.transpose(ti, (0, 2, 1))
    topv = jnp.transpose(tv, (0, 2, 1))

    counts = cnt[0].astype(jnp.int32)
    padded = ((counts + TSG - 1) // TSG) * TSG
    pend = jnp.cumsum(padded)
    pstart = pend - padded
    ppos_bkt = pstart[ti] + rk.astype(jnp.int32)
    ppos = jnp.transpose(ppos_bkt, (0, 2, 1)).reshape(NP)
    tok = (jnp.arange(NP, dtype=jnp.int32) // K)
    tok_pad = (jnp.arange(PT, dtype=jnp.int32) % N).at[ppos].set(tok)
    tile_start = jnp.arange(NT, dtype=jnp.int32) * TSG
    tile_expert = jnp.minimum(
        jnp.sum((tile_start[:, None] >= pend[None, :]).astype(jnp.int32),
                axis=1), E - 1).astype(jnp.int32)
    p12 = ppos.reshape(N, K)
    p1 = p12[:, 0]
    p2 = p12[:, 1]

    xt32 = xt32.reshape(N, CINW)
    xg32 = _gather_x(tok_pad, xt32)
    ewp = (expert_w + res_w[None]).astype(jnp.bfloat16)
    yg32 = _grouped_mm(tile_expert, xg32, ewp, expert_b)
    o1, o2 = _gather_y(p1, p2, yg32)

    out = _epilogue(res_b, tv, o1, o2)
    return out, (topi, topv)


def kernel(x, router_w, expert_w, expert_b, res_w, res_b):
    return _run(x, router_w, expert_w, expert_b, res_w, res_b)


# speedup vs baseline: 1.4074x; 1.4074x over previous
import functools

import jax
import jax.numpy as jnp
from jax import lax
from jax.experimental import pallas as pl
from jax.experimental.pallas import tpu as pltpu
from jax.experimental.pallas import tpu_sc as plsc

B, CIN, COUT, T, E, K = 2, 768, 1024, 2048, 8, 2
N = B * T
NP = N * K
TSG = 512
PT = NP + E * TSG
NT = PT // TSG
TT = 512
NW = 32
CINW = CIN // 2
COUTW = COUT // 2


def _router_body(x_ref, rw_ref, ti_ref, tv_ref, xt_ref, rk_ref, cnt_ref,
                 acc_ref):
    x = x_ref[0]
    logits = jax.lax.dot_general(
        x, rw_ref[...], (((0,), (0,)), ((), ())),
        preferred_element_type=jnp.float32)
    m = jnp.max(logits, axis=-1, keepdims=True)
    p = jnp.exp(logits - m)
    probs = p / jnp.sum(p, axis=-1, keepdims=True)
    eidx = jax.lax.broadcasted_iota(jnp.int32, (TT, E), 1)
    v1 = jnp.max(probs, axis=-1)
    i1 = jnp.min(jnp.where(probs == v1[:, None], eidx, E), axis=-1)
    probs2 = jnp.where(eidx == i1[:, None], -1.0, probs)
    v2 = jnp.max(probs2, axis=-1)
    i2 = jnp.min(jnp.where(probs2 == v2[:, None], eidx, E), axis=-1)
    s = v1 + v2
    ti_ref[0] = jnp.stack([i1, i2])
    tv_ref[0] = jnp.stack([v1 / s, v2 / s])
    xt_ref[0] = pltpu.bitcast(x.astype(jnp.bfloat16), jnp.float32).T

    first = (pl.program_id(0) == 0) & (pl.program_id(1) == 0)

    @pl.when(first)
    def _():
        acc_ref[...] = jnp.zeros((1, E), jnp.float32)

    cnt = acc_ref[0]
    tri = (jax.lax.broadcasted_iota(jnp.int32, (TT, TT), 0)
           > jax.lax.broadcasted_iota(jnp.int32, (TT, TT), 1)
           ).astype(jnp.float32)
    oneh1 = (eidx == i1[:, None]).astype(jnp.float32)
    ex1 = jax.lax.dot_general(tri, oneh1, (((1,), (0,)), ((), ())),
                              preferred_element_type=jnp.float32)
    rank1 = (jnp.sum(ex1 * oneh1, axis=1)
             + jnp.sum(oneh1 * cnt[None, :], axis=1))
    cnt1 = cnt + jnp.sum(oneh1, axis=0)
    oneh2 = (eidx == i2[:, None]).astype(jnp.float32)
    ex2 = jax.lax.dot_general(tri, oneh2, (((1,), (0,)), ((), ())),
                              preferred_element_type=jnp.float32)
    rank2 = (jnp.sum(ex2 * oneh2, axis=1)
             + jnp.sum(oneh2 * cnt1[None, :], axis=1))
    acc_ref[...] = (cnt1 + jnp.sum(oneh2, axis=0))[None, :]
    rk_ref[0] = jnp.stack([rank1, rank2])
    cnt_ref[...] = acc_ref[...]


def _router(x, router_w):
    return pl.pallas_call(
        _router_body,
        grid=(B, T // TT),
        in_specs=[
            pl.BlockSpec((1, CIN, TT), lambda b, t: (b, 0, t)),
            pl.BlockSpec((CIN, E), lambda b, t: (0, 0)),
        ],
        out_specs=[
            pl.BlockSpec((1, K, TT), lambda b, t: (b, 0, t)),
            pl.BlockSpec((1, K, TT), lambda b, t: (b, 0, t)),
            pl.BlockSpec((1, TT, CINW), lambda b, t: (b, t, 0)),
            pl.BlockSpec((1, K, TT), lambda b, t: (b, 0, t)),
            pl.BlockSpec((1, E), lambda b, t: (0, 0)),
        ],
        out_shape=[
            jax.ShapeDtypeStruct((B, K, T), jnp.int32),
            jax.ShapeDtypeStruct((B, K, T), jnp.float32),
            jax.ShapeDtypeStruct((B, T, CINW), jnp.float32),
            jax.ShapeDtypeStruct((B, K, T), jnp.float32),
            jax.ShapeDtypeStruct((1, E), jnp.float32),
        ],
        scratch_shapes=[pltpu.VMEM((1, E), jnp.float32)],
    )(x, router_w)


_RPW2 = PT // NW
_CH2 = 48
_RPW4 = N // NW
_CH4 = 32
_NBUF = 6
_LAG = 4


def _sc_mesh():
    return plsc.VectorSubcoreMesh(core_axis_name="c", subcore_axis_name="s")


def _ring_gather(tasks, table_hbm, idx_v, row_bufs, gsems, wsems, chunk):
    nt = len(tasks)
    gd = [None] * nt
    wd = [None] * nt

    def fire_wb(u):
        gd[u].wait()
        _, out_hbm, ooff = tasks[u]
        wd[u] = pltpu.async_copy(row_bufs[u % _NBUF],
                                 out_hbm.at[pl.ds(ooff, chunk)],
                                 wsems[u % _NBUF])

    for t in range(nt):
        b = t % _NBUF
        if t >= _NBUF:
            wd[t - _NBUF].wait()
        ioff, _, _ = tasks[t]
        gd[t] = pltpu.async_copy(
            table_hbm.at[idx_v.at[pl.ds(ioff, chunk)]], row_bufs[b], gsems[b])
        if t >= _LAG:
            fire_wb(t - _LAG)
    for u in range(max(0, nt - _LAG), nt):
        fire_wb(u)
    for u in range(max(0, nt - _NBUF), nt):
        wd[u].wait()


def _gather_x(tok_pad, xt32):
    @functools.partial(
        pl.kernel, mesh=_sc_mesh(),
        out_type=jax.ShapeDtypeStruct((PT, CINW), jnp.float32),
        scratch_types=(
            [pltpu.VMEM((_RPW2,), jnp.int32)]
            + [pltpu.VMEM((_CH2, CINW), jnp.float32)] * _NBUF
            + [pltpu.SemaphoreType.DMA] * (2 * _NBUF)
        ),
    )
    def k(tok_hbm, xt_hbm, out_hbm, idx_v, r0, r1, r2, r3, r4, r5,
          g0, g1, g2, g3, g4, g5, w0, w1, w2, w3, w4, w5):
        wid = lax.axis_index("s") * 2 + lax.axis_index("c")
        base = wid * _RPW2
        pltpu.sync_copy(tok_hbm.at[pl.ds(base, _RPW2)], idx_v)
        tasks = [(c * _CH2, out_hbm, base + c * _CH2)
                 for c in range(_RPW2 // _CH2)]
        _ring_gather(tasks, xt_hbm, idx_v, [r0, r1, r2, r3, r4, r5],
                     [g0, g1, g2, g3, g4, g5], [w0, w1, w2, w3, w4, w5],
                     _CH2)

    return k(tok_pad, xt32)


def _gather_y(p1, p2, yg32):
    @functools.partial(
        pl.kernel, mesh=_sc_mesh(),
        out_type=[jax.ShapeDtypeStruct((N, COUTW), jnp.float32),
                  jax.ShapeDtypeStruct((N, COUTW), jnp.float32)],
        scratch_types=(
            [pltpu.VMEM((2 * _RPW4,), jnp.int32)]
            + [pltpu.VMEM((_CH4, COUTW), jnp.float32)] * _NBUF
            + [pltpu.SemaphoreType.DMA] * (2 * _NBUF)
        ),
    )
    def k(p1_hbm, p2_hbm, yg_hbm, o1_hbm, o2_hbm, idx_v,
          r0, r1, r2, r3, r4, r5,
          g0, g1, g2, g3, g4, g5, w0, w1, w2, w3, w4, w5):
        wid = lax.axis_index("s") * 2 + lax.axis_index("c")
        base = wid * _RPW4
        pltpu.sync_copy(p1_hbm.at[pl.ds(base, _RPW4)],
                        idx_v.at[pl.ds(0, _RPW4)])
        pltpu.sync_copy(p2_hbm.at[pl.ds(base, _RPW4)],
                        idx_v.at[pl.ds(_RPW4, _RPW4)])
        nch = _RPW4 // _CH4
        tasks = ([(c * _CH4, o1_hbm, base + c * _CH4) for c in range(nch)]
                 + [(_RPW4 + c * _CH4, o2_hbm, base + c * _CH4)
                    for c in range(nch)])
        _ring_gather(tasks, yg_hbm, idx_v, [r0, r1, r2, r3, r4, r5],
                     [g0, g1, g2, g3, g4, g5], [w0, w1, w2, w3, w4, w5],
                     _CH4)

    return k(p1, p2, yg32)


def _mm_body(te_ref, xg_ref, ew_ref, eb_ref, yg_ref):
    xb = pltpu.bitcast(xg_ref[...].T, jnp.bfloat16)
    y = jax.lax.dot_general(
        ew_ref[0], xb, (((1,), (0,)), ((), ())),
        preferred_element_type=jnp.float32)
    y = y + eb_ref[0][0][:, None]
    yb = y.astype(jnp.bfloat16)
    yg_ref[...] = pltpu.bitcast(yb, jnp.float32).T


def _grouped_mm(tile_expert, xg32, ew_b, expert_b):
    grid_spec = pltpu.PrefetchScalarGridSpec(
        num_scalar_prefetch=1,
        grid=(NT,),
        in_specs=[
            pl.BlockSpec((TSG, CINW), lambda g, te: (g, 0)),
            pl.BlockSpec((1, COUT, CIN), lambda g, te: (te[g], 0, 0)),
            pl.BlockSpec((1, 1, COUT), lambda g, te: (te[g], 0, 0)),
        ],
        out_specs=pl.BlockSpec((TSG, COUTW), lambda g, te: (g, 0)),
    )
    return pl.pallas_call(
        _mm_body,
        grid_spec=grid_spec,
        out_shape=jax.ShapeDtypeStruct((PT, COUTW), jnp.float32),
    )(tile_expert, xg32, ew_b, expert_b.reshape(E, 1, COUT))


def _ep_body(resb_ref, tv_ref, o1_ref, o2_ref, out_ref):
    y1 = pltpu.bitcast(o1_ref[...].T, jnp.bfloat16)
    y2 = pltpu.bitcast(o2_ref[...].T, jnp.bfloat16)
    v = tv_ref[0]
    acc = resb_ref[0][:, None] + (
        y1.astype(jnp.float32) * v[0][None, :]
        + y2.astype(jnp.float32) * v[1][None, :])
    out_ref[0] = acc


def _epilogue(res_b, tv, o1, o2):
    nt = T // TT
    return pl.pallas_call(
        _ep_body,
        grid=(B, nt),
        in_specs=[
            pl.BlockSpec((1, COUT), lambda b, t: (0, 0)),
            pl.BlockSpec((1, K, TT), lambda b, t: (b, 0, t)),
            pl.BlockSpec((TT, COUTW), lambda b, t: (b * nt + t, 0)),
            pl.BlockSpec((TT, COUTW), lambda b, t: (b * nt + t, 0)),
        ],
        out_specs=pl.BlockSpec((1, COUT, TT), lambda b, t: (b, 0, t)),
        out_shape=jax.ShapeDtypeStruct((B, COUT, T), jnp.float32),
    )(res_b.reshape(1, COUT), tv, o1, o2)


@jax.jit
def _run(x, router_w, expert_w, expert_b, res_w, res_b):
    ti, tv, xt32, rk, cnt = _router(x, router_w)
    topi = jnp.transpose(ti, (0, 2, 1))
    topv = jnp.transpose(tv, (0, 2, 1))

    counts = cnt[0].astype(jnp.int32)
    padded = ((counts + TSG - 1) // TSG) * TSG
    pend = jnp.cumsum(padded)
    pstart = pend - padded
    pstart_sel = jnp.sum(
        pstart[None, None, None, :]
        * (ti[..., None] == jnp.arange(E, dtype=jnp.int32)).astype(jnp.int32),
        axis=-1)
    ppos_bkt = pstart_sel + rk.astype(jnp.int32)
    ppos = jnp.transpose(ppos_bkt, (0, 2, 1)).reshape(NP)
    tok = (jnp.arange(NP, dtype=jnp.int32) // K)
    tok_pad = (jnp.arange(PT, dtype=jnp.int32) % N).at[ppos].set(tok)
    tile_start = jnp.arange(NT, dtype=jnp.int32) * TSG
    tile_expert = jnp.minimum(
        jnp.sum((tile_start[:, None] >= pend[None, :]).astype(jnp.int32),
                axis=1), E - 1).astype(jnp.int32)
    p12 = ppos.reshape(N, K)
    p1 = p12[:, 0]
    p2 = p12[:, 1]

    xt32 = xt32.reshape(N, CINW)
    xg32 = _gather_x(tok_pad, xt32)
    ewp = (expert_w + res_w[None]).astype(jnp.bfloat16)
    yg32 = _grouped_mm(tile_expert, xg32, ewp, expert_b)
    o1, o2 = _gather_y(p1, p2, yg32)

    out = _epilogue(res_b, tv, o1, o2)
    return out, (topi, topv)


def kernel(x, router_w, expert_w, expert_b, res_w, res_b):
    return _run(x, router_w, expert_w, expert_b, res_w, res_b)


# speedup vs baseline: 1.8081x; 1.2847x over previous
import functools

import jax
import jax.numpy as jnp
from jax import lax
from jax.experimental import pallas as pl
from jax.experimental.pallas import tpu as pltpu
from jax.experimental.pallas import tpu_sc as plsc

B, CIN, COUT, T, E, K = 2, 768, 1024, 2048, 8, 2
N = B * T
NP = N * K
TSG = 512
PT = NP + E * TSG
NT = PT // TSG
TT = 512
NW = 32
CINW = CIN // 2
COUTW = COUT // 2


def _router_body(x_ref, rw_ref, ti_ref, tv_ref, xt_ref, rk_ref, cnt_ref,
                 acc_ref):
    x = x_ref[0]
    logits = jax.lax.dot_general(
        x, rw_ref[...], (((0,), (0,)), ((), ())),
        preferred_element_type=jnp.float32)
    m = jnp.max(logits, axis=-1, keepdims=True)
    p = jnp.exp(logits - m)
    probs = p / jnp.sum(p, axis=-1, keepdims=True)
    eidx = jax.lax.broadcasted_iota(jnp.int32, (TT, E), 1)
    v1 = jnp.max(probs, axis=-1)
    i1 = jnp.min(jnp.where(probs == v1[:, None], eidx, E), axis=-1)
    probs2 = jnp.where(eidx == i1[:, None], -1.0, probs)
    v2 = jnp.max(probs2, axis=-1)
    i2 = jnp.min(jnp.where(probs2 == v2[:, None], eidx, E), axis=-1)
    s = v1 + v2
    ti_ref[0] = jnp.stack([i1, i2])
    tv_ref[0] = jnp.stack([v1 / s, v2 / s])
    xt_ref[0] = pltpu.bitcast(x.astype(jnp.bfloat16), jnp.float32).T

    first = (pl.program_id(0) == 0) & (pl.program_id(1) == 0)

    @pl.when(first)
    def _():
        acc_ref[...] = jnp.zeros((1, E), jnp.float32)

    cnt = acc_ref[0]
    tri = (jax.lax.broadcasted_iota(jnp.int32, (TT, TT), 0)
           > jax.lax.broadcasted_iota(jnp.int32, (TT, TT), 1)
           ).astype(jnp.float32)
    oneh1 = (eidx == i1[:, None]).astype(jnp.float32)
    ex1 = jax.lax.dot_general(tri, oneh1, (((1,), (0,)), ((), ())),
                              preferred_element_type=jnp.float32)
    rank1 = (jnp.sum(ex1 * oneh1, axis=1)
             + jnp.sum(oneh1 * cnt[None, :], axis=1))
    cnt1 = cnt + jnp.sum(oneh1, axis=0)
    oneh2 = (eidx == i2[:, None]).astype(jnp.float32)
    ex2 = jax.lax.dot_general(tri, oneh2, (((1,), (0,)), ((), ())),
                              preferred_element_type=jnp.float32)
    rank2 = (jnp.sum(ex2 * oneh2, axis=1)
             + jnp.sum(oneh2 * cnt1[None, :], axis=1))
    acc_ref[...] = (cnt1 + jnp.sum(oneh2, axis=0))[None, :]
    rk_ref[0] = jnp.stack([rank1, rank2])
    cnt_ref[...] = acc_ref[...]


def _router(x, router_w):
    return pl.pallas_call(
        _router_body,
        grid=(B, T // TT),
        in_specs=[
            pl.BlockSpec((1, CIN, TT), lambda b, t: (b, 0, t)),
            pl.BlockSpec((CIN, E), lambda b, t: (0, 0)),
        ],
        out_specs=[
            pl.BlockSpec((1, K, TT), lambda b, t: (b, 0, t)),
            pl.BlockSpec((1, K, TT), lambda b, t: (b, 0, t)),
            pl.BlockSpec((1, TT, CINW), lambda b, t: (b, t, 0)),
            pl.BlockSpec((1, K, TT), lambda b, t: (b, 0, t)),
            pl.BlockSpec((1, E), lambda b, t: (0, 0)),
        ],
        out_shape=[
            jax.ShapeDtypeStruct((B, K, T), jnp.int32),
            jax.ShapeDtypeStruct((B, K, T), jnp.float32),
            jax.ShapeDtypeStruct((B, T, CINW), jnp.float32),
            jax.ShapeDtypeStruct((B, K, T), jnp.float32),
            jax.ShapeDtypeStruct((1, E), jnp.float32),
        ],
        scratch_shapes=[pltpu.VMEM((1, E), jnp.float32)],
    )(x, router_w)


_RPW2 = PT // NW
_CH2 = 48
_RPW4 = N // NW
_CH4 = 32
_NBUF = 6
_LAG = 4


def _sc_mesh():
    return plsc.VectorSubcoreMesh(core_axis_name="c", subcore_axis_name="s")


def _ring_gather(tasks, table_hbm, idx_v, row_bufs, gsems, wsems, chunk):
    nt = len(tasks)
    gd = [None] * nt
    wd = [None] * nt

    def fire_wb(u):
        gd[u].wait()
        _, out_hbm, ooff = tasks[u]
        wd[u] = pltpu.async_copy(row_bufs[u % _NBUF],
                                 out_hbm.at[pl.ds(ooff, chunk)],
                                 wsems[u % _NBUF])

    for t in range(nt):
        b = t % _NBUF
        if t >= _NBUF:
            wd[t - _NBUF].wait()
        ioff, _, _ = tasks[t]
        gd[t] = pltpu.async_copy(
            table_hbm.at[idx_v.at[pl.ds(ioff, chunk)]], row_bufs[b], gsems[b])
        if t >= _LAG:
            fire_wb(t - _LAG)
    for u in range(max(0, nt - _LAG), nt):
        fire_wb(u)
    for u in range(max(0, nt - _NBUF), nt):
        wd[u].wait()


_TPW = N // NW
_CHX = 32


def _scatter_x(p1, p2, xt32):
    @functools.partial(
        pl.kernel, mesh=_sc_mesh(),
        out_type=jax.ShapeDtypeStruct((PT, CINW), jnp.float32),
        scratch_types=(
            [pltpu.VMEM((2 * _TPW,), jnp.int32)]
            + [pltpu.VMEM((_CHX, CINW), jnp.float32)] * _NBUF
            + [pltpu.SemaphoreType.DMA] * (2 * _NBUF)
        ),
    )
    def k(p1_hbm, p2_hbm, xt_hbm, out_hbm, idx_v, r0, r1, r2, r3, r4, r5,
          g0, g1, g2, g3, g4, g5, w0, w1, w2, w3, w4, w5):
        wid = lax.axis_index("s") * 2 + lax.axis_index("c")
        base = wid * _TPW
        pltpu.sync_copy(p1_hbm.at[pl.ds(base, _TPW)],
                        idx_v.at[pl.ds(0, _TPW)])
        pltpu.sync_copy(p2_hbm.at[pl.ds(base, _TPW)],
                        idx_v.at[pl.ds(_TPW, _TPW)])
        bufs = [r0, r1, r2, r3, r4, r5]
        gsems = [g0, g1, g2, g3, g4, g5]
        wsems = [w0, w1, w2, w3, w4, w5]
        nch = _TPW // _CHX
        gd = [None] * nch
        for c in range(nch):
            gd[c] = pltpu.async_copy(
                xt_hbm.at[pl.ds(base + c * _CHX, _CHX)], bufs[c], gsems[c])
        wds = [None] * _NBUF
        si = 0
        for c in range(nch):
            gd[c].wait()
            for ioff in (c * _CHX, _TPW + c * _CHX):
                if wds[si] is not None:
                    wds[si].wait()
                wds[si] = pltpu.async_copy(
                    bufs[c], out_hbm.at[idx_v.at[pl.ds(ioff, _CHX)]],
                    wsems[si])
                si = (si + 1) % _NBUF
        for w in wds:
            if w is not None:
                w.wait()

    return k(p1, p2, xt32)


def _gather_y(p1, p2, yg32):
    @functools.partial(
        pl.kernel, mesh=_sc_mesh(),
        out_type=[jax.ShapeDtypeStruct((N, COUTW), jnp.float32),
                  jax.ShapeDtypeStruct((N, COUTW), jnp.float32)],
        scratch_types=(
            [pltpu.VMEM((2 * _RPW4,), jnp.int32)]
            + [pltpu.VMEM((_CH4, COUTW), jnp.float32)] * _NBUF
            + [pltpu.SemaphoreType.DMA] * (2 * _NBUF)
        ),
    )
    def k(p1_hbm, p2_hbm, yg_hbm, o1_hbm, o2_hbm, idx_v,
          r0, r1, r2, r3, r4, r5,
          g0, g1, g2, g3, g4, g5, w0, w1, w2, w3, w4, w5):
        wid = lax.axis_index("s") * 2 + lax.axis_index("c")
        base = wid * _RPW4
        pltpu.sync_copy(p1_hbm.at[pl.ds(base, _RPW4)],
                        idx_v.at[pl.ds(0, _RPW4)])
        pltpu.sync_copy(p2_hbm.at[pl.ds(base, _RPW4)],
                        idx_v.at[pl.ds(_RPW4, _RPW4)])
        nch = _RPW4 // _CH4
        tasks = ([(c * _CH4, o1_hbm, base + c * _CH4) for c in range(nch)]
                 + [(_RPW4 + c * _CH4, o2_hbm, base + c * _CH4)
                    for c in range(nch)])
        _ring_gather(tasks, yg_hbm, idx_v, [r0, r1, r2, r3, r4, r5],
                     [g0, g1, g2, g3, g4, g5], [w0, w1, w2, w3, w4, w5],
                     _CH4)

    return k(p1, p2, yg32)


def _mm_body(te_ref, xg_ref, ew_ref, eb_ref, yg_ref):
    xb = pltpu.bitcast(xg_ref[...].T, jnp.bfloat16)
    y = jax.lax.dot_general(
        ew_ref[0], xb, (((1,), (0,)), ((), ())),
        preferred_element_type=jnp.float32)
    y = y + eb_ref[0][0][:, None]
    yb = y.astype(jnp.bfloat16)
    yg_ref[...] = pltpu.bitcast(yb, jnp.float32).T


def _grouped_mm(tile_expert, xg32, ew_b, expert_b):
    grid_spec = pltpu.PrefetchScalarGridSpec(
        num_scalar_prefetch=1,
        grid=(NT,),
        in_specs=[
            pl.BlockSpec((TSG, CINW), lambda g, te: (g, 0)),
            pl.BlockSpec((1, COUT, CIN), lambda g, te: (te[g], 0, 0)),
            pl.BlockSpec((1, 1, COUT), lambda g, te: (te[g], 0, 0)),
        ],
        out_specs=pl.BlockSpec((TSG, COUTW), lambda g, te: (g, 0)),
    )
    return pl.pallas_call(
        _mm_body,
        grid_spec=grid_spec,
        out_shape=jax.ShapeDtypeStruct((PT, COUTW), jnp.float32),
    )(tile_expert, xg32, ew_b, expert_b.reshape(E, 1, COUT))


def _ep_body(resb_ref, tv_ref, o1_ref, o2_ref, out_ref):
    y1 = pltpu.bitcast(o1_ref[...].T, jnp.bfloat16)
    y2 = pltpu.bitcast(o2_ref[...].T, jnp.bfloat16)
    v = tv_ref[0]
    acc = resb_ref[0][:, None] + (
        y1.astype(jnp.float32) * v[0][None, :]
        + y2.astype(jnp.float32) * v[1][None, :])
    out_ref[0] = acc


def _epilogue(res_b, tv, o1, o2):
    nt = T // TT
    return pl.pallas_call(
        _ep_body,
        grid=(B, nt),
        in_specs=[
            pl.BlockSpec((1, COUT), lambda b, t: (0, 0)),
            pl.BlockSpec((1, K, TT), lambda b, t: (b, 0, t)),
            pl.BlockSpec((TT, COUTW), lambda b, t: (b * nt + t, 0)),
            pl.BlockSpec((TT, COUTW), lambda b, t: (b * nt + t, 0)),
        ],
        out_specs=pl.BlockSpec((1, COUT, TT), lambda b, t: (b, 0, t)),
        out_shape=jax.ShapeDtypeStruct((B, COUT, T), jnp.float32),
    )(res_b.reshape(1, COUT), tv, o1, o2)


@jax.jit
def _run(x, router_w, expert_w, expert_b, res_w, res_b):
    ti, tv, xt32, rk, cnt = _router(x, router_w)
    topi = jnp.transpose(ti, (0, 2, 1))
    topv = jnp.transpose(tv, (0, 2, 1))

    counts = cnt[0].astype(jnp.int32)
    padded = ((counts + TSG - 1) // TSG) * TSG
    pend = jnp.cumsum(padded)
    pstart = pend - padded
    pstart_sel = jnp.sum(
        pstart[None, None, None, :]
        * (ti[..., None] == jnp.arange(E, dtype=jnp.int32)).astype(jnp.int32),
        axis=-1)
    ppos_bkt = pstart_sel + rk.astype(jnp.int32)
    tile_start = jnp.arange(NT, dtype=jnp.int32) * TSG
    tile_expert = jnp.minimum(
        jnp.sum((tile_start[:, None] >= pend[None, :]).astype(jnp.int32),
                axis=1), E - 1).astype(jnp.int32)
    p1 = ppos_bkt[:, 0, :].reshape(N)
    p2 = ppos_bkt[:, 1, :].reshape(N)

    xt32 = xt32.reshape(N, CINW)
    xg32 = _scatter_x(p1, p2, xt32)
    ewp = (expert_w + res_w[None]).astype(jnp.bfloat16)
    yg32 = _grouped_mm(tile_expert, xg32, ewp, expert_b)
    o1, o2 = _gather_y(p1, p2, yg32)

    out = _epilogue(res_b, tv, o1, o2)
    return out, (topi, topv)


def kernel(x, router_w, expert_w, expert_b, res_w, res_b):
    return _run(x, router_w, expert_w, expert_b, res_w, res_b)


# speedup vs baseline: 1.8974x; 1.0494x over previous
import functools

import jax
import jax.numpy as jnp
from jax import lax
from jax.experimental import pallas as pl
from jax.experimental.pallas import tpu as pltpu
from jax.experimental.pallas import tpu_sc as plsc

B, CIN, COUT, T, E, K = 2, 768, 1024, 2048, 8, 2
N = B * T
NP = N * K
TSG = 512
PT = NP + E * TSG
NT = PT // TSG
TT = 512
NW = 32
CINW = CIN // 2
COUTW = COUT // 2


def _router_body(x_ref, rw_ref, ti_ref, tv_ref, xt_ref, rk_ref, cnt_ref,
                 acc_ref):
    x = x_ref[0]
    logits = jax.lax.dot_general(
        x, rw_ref[...], (((0,), (0,)), ((), ())),
        preferred_element_type=jnp.float32)
    m = jnp.max(logits, axis=-1, keepdims=True)
    p = jnp.exp(logits - m)
    probs = p / jnp.sum(p, axis=-1, keepdims=True)
    eidx = jax.lax.broadcasted_iota(jnp.int32, (TT, E), 1)
    v1 = jnp.max(probs, axis=-1)
    i1 = jnp.min(jnp.where(probs == v1[:, None], eidx, E), axis=-1)
    probs2 = jnp.where(eidx == i1[:, None], -1.0, probs)
    v2 = jnp.max(probs2, axis=-1)
    i2 = jnp.min(jnp.where(probs2 == v2[:, None], eidx, E), axis=-1)
    s = v1 + v2
    ti_ref[0] = jnp.stack([i1, i2])
    tv_ref[0] = jnp.stack([v1 / s, v2 / s])
    xt_ref[0] = pltpu.bitcast(x.astype(jnp.bfloat16), jnp.float32).T

    first = (pl.program_id(0) == 0) & (pl.program_id(1) == 0)

    @pl.when(first)
    def _():
        acc_ref[...] = jnp.zeros((1, E), jnp.float32)

    cnt = acc_ref[0]
    tri = (jax.lax.broadcasted_iota(jnp.int32, (TT, TT), 0)
           > jax.lax.broadcasted_iota(jnp.int32, (TT, TT), 1)
           ).astype(jnp.float32)
    oneh1 = (eidx == i1[:, None]).astype(jnp.float32)
    ex1 = jax.lax.dot_general(tri, oneh1, (((1,), (0,)), ((), ())),
                              preferred_element_type=jnp.float32)
    rank1 = (jnp.sum(ex1 * oneh1, axis=1)
             + jnp.sum(oneh1 * cnt[None, :], axis=1))
    cnt1 = cnt + jnp.sum(oneh1, axis=0)
    oneh2 = (eidx == i2[:, None]).astype(jnp.float32)
    ex2 = jax.lax.dot_general(tri, oneh2, (((1,), (0,)), ((), ())),
                              preferred_element_type=jnp.float32)
    rank2 = (jnp.sum(ex2 * oneh2, axis=1)
             + jnp.sum(oneh2 * cnt1[None, :], axis=1))
    acc_ref[...] = (cnt1 + jnp.sum(oneh2, axis=0))[None, :]
    rk_ref[0] = jnp.stack([rank1, rank2])
    cnt_ref[...] = acc_ref[...]


def _router(x, router_w):
    return pl.pallas_call(
        _router_body,
        grid=(B, T // TT),
        in_specs=[
            pl.BlockSpec((1, CIN, TT), lambda b, t: (b, 0, t)),
            pl.BlockSpec((CIN, E), lambda b, t: (0, 0)),
        ],
        out_specs=[
            pl.BlockSpec((1, K, TT), lambda b, t: (b, 0, t)),
            pl.BlockSpec((1, K, TT), lambda b, t: (b, 0, t)),
            pl.BlockSpec((1, TT, CINW), lambda b, t: (b, t, 0)),
            pl.BlockSpec((1, K, TT), lambda b, t: (b, 0, t)),
            pl.BlockSpec((1, E), lambda b, t: (0, 0)),
        ],
        out_shape=[
            jax.ShapeDtypeStruct((B, K, T), jnp.int32),
            jax.ShapeDtypeStruct((B, K, T), jnp.float32),
            jax.ShapeDtypeStruct((B, T, CINW), jnp.float32),
            jax.ShapeDtypeStruct((B, K, T), jnp.float32),
            jax.ShapeDtypeStruct((1, E), jnp.float32),
        ],
        scratch_shapes=[pltpu.VMEM((1, E), jnp.float32)],
    )(x, router_w)


_RPW2 = PT // NW
_CH2 = 48
_RPW4 = N // NW
_CH4 = 32
_NBUF = 6
_LAG = 4


def _sc_mesh():
    return plsc.VectorSubcoreMesh(core_axis_name="c", subcore_axis_name="s")


def _ring_gather(tasks, table_hbm, idx_v, row_bufs, gsems, wsems, chunk):
    nt = len(tasks)
    gd = [None] * nt
    wd = [None] * nt

    def fire_wb(u):
        gd[u].wait()
        _, out_hbm, ooff = tasks[u]
        wd[u] = pltpu.async_copy(row_bufs[u % _NBUF],
                                 out_hbm.at[pl.ds(ooff, chunk)],
                                 wsems[u % _NBUF])

    for t in range(nt):
        b = t % _NBUF
        if t >= _NBUF:
            wd[t - _NBUF].wait()
        ioff, _, _ = tasks[t]
        gd[t] = pltpu.async_copy(
            table_hbm.at[idx_v.at[pl.ds(ioff, chunk)]], row_bufs[b], gsems[b])
        if t >= _LAG:
            fire_wb(t - _LAG)
    for u in range(max(0, nt - _LAG), nt):
        fire_wb(u)
    for u in range(max(0, nt - _NBUF), nt):
        wd[u].wait()


_TPW = N // NW
_CHX = 32


def _scatter_x(p1, p2, xt32):
    @functools.partial(
        pl.kernel, mesh=_sc_mesh(),
        out_type=jax.ShapeDtypeStruct((PT, CINW), jnp.float32),
        scratch_types=(
            [pltpu.VMEM((2 * _TPW,), jnp.int32)]
            + [pltpu.VMEM((_CHX, CINW), jnp.float32)] * _NBUF
            + [pltpu.SemaphoreType.DMA] * (2 * _NBUF)
        ),
    )
    def k(p1_hbm, p2_hbm, xt_hbm, out_hbm, idx_v, r0, r1, r2, r3, r4, r5,
          g0, g1, g2, g3, g4, g5, w0, w1, w2, w3, w4, w5):
        wid = lax.axis_index("s") * 2 + lax.axis_index("c")
        base = wid * _TPW
        pltpu.sync_copy(p1_hbm.at[pl.ds(base, _TPW)],
                        idx_v.at[pl.ds(0, _TPW)])
        pltpu.sync_copy(p2_hbm.at[pl.ds(base, _TPW)],
                        idx_v.at[pl.ds(_TPW, _TPW)])
        bufs = [r0, r1, r2, r3, r4, r5]
        gsems = [g0, g1, g2, g3, g4, g5]
        wsems = [w0, w1, w2, w3, w4, w5]
        nch = _TPW // _CHX
        gd = [None] * nch
        for c in range(nch):
            gd[c] = pltpu.async_copy(
                xt_hbm.at[pl.ds(base + c * _CHX, _CHX)], bufs[c], gsems[c])
        wds = [None] * _NBUF
        si = 0
        for c in range(nch):
            gd[c].wait()
            for ioff in (c * _CHX, _TPW + c * _CHX):
                if wds[si] is not None:
                    wds[si].wait()
                wds[si] = pltpu.async_copy(
                    bufs[c], out_hbm.at[idx_v.at[pl.ds(ioff, _CHX)]],
                    wsems[si])
                si = (si + 1) % _NBUF
        for w in wds:
            if w is not None:
                w.wait()

    return k(p1, p2, xt32)


def _gather_y(p1, p2, yg32):
    @functools.partial(
        pl.kernel, mesh=_sc_mesh(),
        out_type=[jax.ShapeDtypeStruct((N, COUTW), jnp.float32),
                  jax.ShapeDtypeStruct((N, COUTW), jnp.float32)],
        scratch_types=(
            [pltpu.VMEM((2 * _RPW4,), jnp.int32)]
            + [pltpu.VMEM((_CH4, COUTW), jnp.float32)] * _NBUF
            + [pltpu.SemaphoreType.DMA] * (2 * _NBUF)
        ),
    )
    def k(p1_hbm, p2_hbm, yg_hbm, o1_hbm, o2_hbm, idx_v,
          r0, r1, r2, r3, r4, r5,
          g0, g1, g2, g3, g4, g5, w0, w1, w2, w3, w4, w5):
        wid = lax.axis_index("s") * 2 + lax.axis_index("c")
        base = wid * _RPW4
        pltpu.sync_copy(p1_hbm.at[pl.ds(base, _RPW4)],
                        idx_v.at[pl.ds(0, _RPW4)])
        pltpu.sync_copy(p2_hbm.at[pl.ds(base, _RPW4)],
                        idx_v.at[pl.ds(_RPW4, _RPW4)])
        nch = _RPW4 // _CH4
        tasks = ([(c * _CH4, o1_hbm, base + c * _CH4) for c in range(nch)]
                 + [(_RPW4 + c * _CH4, o2_hbm, base + c * _CH4)
                    for c in range(nch)])
        _ring_gather(tasks, yg_hbm, idx_v, [r0, r1, r2, r3, r4, r5],
                     [g0, g1, g2, g3, g4, g5], [w0, w1, w2, w3, w4, w5],
                     _CH4)

    return k(p1, p2, yg32)


def _mm_body(te_ref, xg_ref, ew_ref, rw_ref, eb_ref, yg_ref):
    xb = pltpu.bitcast(xg_ref[...].T, jnp.bfloat16)
    w = (ew_ref[0] + rw_ref[...]).astype(jnp.bfloat16)
    y = jax.lax.dot_general(
        w, xb, (((1,), (0,)), ((), ())),
        preferred_element_type=jnp.float32)
    y = y + eb_ref[0][0][:, None]
    yb = y.astype(jnp.bfloat16)
    yg_ref[...] = pltpu.bitcast(yb, jnp.float32).T


def _grouped_mm(tile_expert, xg32, expert_w, res_w, expert_b):
    grid_spec = pltpu.PrefetchScalarGridSpec(
        num_scalar_prefetch=1,
        grid=(NT,),
        in_specs=[
            pl.BlockSpec((TSG, CINW), lambda g, te: (g, 0)),
            pl.BlockSpec((1, COUT, CIN), lambda g, te: (te[g], 0, 0)),
            pl.BlockSpec((COUT, CIN), lambda g, te: (0, 0)),
            pl.BlockSpec((1, 1, COUT), lambda g, te: (te[g], 0, 0)),
        ],
        out_specs=pl.BlockSpec((TSG, COUTW), lambda g, te: (g, 0)),
    )
    return pl.pallas_call(
        _mm_body,
        grid_spec=grid_spec,
        out_shape=jax.ShapeDtypeStruct((PT, COUTW), jnp.float32),
    )(tile_expert, xg32, expert_w, res_w, expert_b.reshape(E, 1, COUT))


def _ep_body(resb_ref, tv_ref, o1_ref, o2_ref, out_ref):
    y1 = pltpu.bitcast(o1_ref[...].T, jnp.bfloat16)
    y2 = pltpu.bitcast(o2_ref[...].T, jnp.bfloat16)
    v = tv_ref[0]
    acc = resb_ref[0][:, None] + (
        y1.astype(jnp.float32) * v[0][None, :]
        + y2.astype(jnp.float32) * v[1][None, :])
    out_ref[0] = acc


def _epilogue(res_b, tv, o1, o2):
    nt = T // TT
    return pl.pallas_call(
        _ep_body,
        grid=(B, nt),
        in_specs=[
            pl.BlockSpec((1, COUT), lambda b, t: (0, 0)),
            pl.BlockSpec((1, K, TT), lambda b, t: (b, 0, t)),
            pl.BlockSpec((TT, COUTW), lambda b, t: (b * nt + t, 0)),
            pl.BlockSpec((TT, COUTW), lambda b, t: (b * nt + t, 0)),
        ],
        out_specs=pl.BlockSpec((1, COUT, TT), lambda b, t: (b, 0, t)),
        out_shape=jax.ShapeDtypeStruct((B, COUT, T), jnp.float32),
    )(res_b.reshape(1, COUT), tv, o1, o2)


@jax.jit
def _run(x, router_w, expert_w, expert_b, res_w, res_b):
    ti, tv, xt32, rk, cnt = _router(x, router_w)
    topi = jnp.transpose(ti, (0, 2, 1))
    topv = jnp.transpose(tv, (0, 2, 1))

    counts = cnt[0].astype(jnp.int32)
    padded = ((counts + TSG - 1) // TSG) * TSG
    pend = jnp.cumsum(padded)
    pstart = pend - padded
    pstart_sel = jnp.sum(
        pstart[None, None, None, :]
        * (ti[..., None] == jnp.arange(E, dtype=jnp.int32)).astype(jnp.int32),
        axis=-1)
    ppos_bkt = pstart_sel + rk.astype(jnp.int32)
    tile_start = jnp.arange(NT, dtype=jnp.int32) * TSG
    tile_expert = jnp.minimum(
        jnp.sum((tile_start[:, None] >= pend[None, :]).astype(jnp.int32),
                axis=1), E - 1).astype(jnp.int32)
    p1 = ppos_bkt[:, 0, :].reshape(N)
    p2 = ppos_bkt[:, 1, :].reshape(N)

    xt32 = xt32.reshape(N, CINW)
    xg32 = _scatter_x(p1, p2, xt32)
    yg32 = _grouped_mm(tile_expert, xg32, expert_w, res_w, expert_b)
    o1, o2 = _gather_y(p1, p2, yg32)

    out = _epilogue(res_b, tv, o1, o2)
    return out, (topi, topv)


def kernel(x, router_w, expert_w, expert_b, res_w, res_b):
    return _run(x, router_w, expert_w, expert_b, res_w, res_b)
